# Initial kernel scaffold; baseline (speedup 1.0000x reference)
import jax
import jax.numpy as jnp
from jax.experimental import pallas as pl


def kernel(x, edge_index, W1, b1, W2, b2):
    raise NotImplementedError("write your pallas kernel here")


# speedup vs baseline: 50.4692x; 50.4692x over previous
import functools

import jax
import jax.numpy as jnp
from jax import lax
from jax.experimental import pallas as pl
from jax.experimental.pallas import tpu as pltpu
from jax.experimental.pallas import tpu_sc as plsc

N = 100000
E = 3200000
NC, NS = 2, 16
NW = NC * NS
LANE = 128
KC = 24
CHUNK = KC * LANE
EPW = ((E + NW * CHUNK - 1) // (NW * CHUNK)) * CHUNK
EP = EPW * NW
NCHUNKS = EPW // CHUNK
NB = 6272
NPAD = NB * NS

_MESH = plsc.VectorSubcoreMesh(core_axis_name="c", subcore_axis_name="s")


def _make_deg():
    @functools.partial(
        pl.kernel,
        out_type=jax.ShapeDtypeStruct((NC, NPAD), jnp.float32),
        mesh=_MESH,
        scratch_types=[
            pltpu.VMEM((CHUNK,), jnp.int32),
            pltpu.VMEM((LANE,), jnp.float32),
            pltpu.VMEM_SHARED((NPAD,), jnp.float32),
        ],
    )
    def deg_kernel(dst_hbm, ones_hbm, zeros_hbm, out_hbm, didx, ones_v, acc):
        c = lax.axis_index("c")
        s = lax.axis_index("s")
        wid = c * NS + s
        pltpu.sync_copy(ones_hbm, ones_v)
        pltpu.sync_copy(zeros_hbm, acc.at[pl.ds(s * NB, NB)])
        plsc.subcore_barrier()
        ebase0 = wid * EPW

        def body(i, carry):
            pltpu.sync_copy(dst_hbm.at[pl.ds(ebase0 + i * CHUNK, CHUNK)], didx)
            for j in range(KC):
                pltpu.sync_copy(ones_v, acc.at[didx.at[pl.ds(j * LANE, LANE)]],
                                add=True)
            return carry

        lax.fori_loop(0, NCHUNKS, body, 0)
        plsc.subcore_barrier()
        pltpu.sync_copy(acc.at[pl.ds(s * NB, NB)],
                        out_hbm.at[c].at[pl.ds(s * NB, NB)])

    return deg_kernel


def _make_prop(C, stage_table, kc):
    chunk = kc * LANE
    nchunks = EPW // chunk
    assert nchunks * chunk == EPW
    scratch = [
        pltpu.VMEM((chunk,), jnp.int32),
        pltpu.VMEM((chunk,), jnp.int32),
        pltpu.VMEM((chunk, C), jnp.float32),
        pltpu.VMEM_SHARED((NPAD, C), jnp.float32),
        pltpu.SemaphoreType.DMA,
    ]
    if stage_table:
        scratch.append(pltpu.VMEM_SHARED((NPAD, C), jnp.float32))

    @functools.partial(
        pl.kernel,
        out_type=jax.ShapeDtypeStruct((NC, NPAD, C), jnp.float32),
        mesh=_MESH,
        scratch_types=scratch,
        compiler_params=pltpu.CompilerParams(use_tc_tiling_on_sc=False),
    )
    def prop_kernel(src_hbm, dst_hbm, table_hbm, zeros_hbm, out_hbm,
                    sidx, didx, rows, acc, sem, *maybe_tab):
        c = lax.axis_index("c")
        s = lax.axis_index("s")
        wid = c * NS + s
        pltpu.sync_copy(zeros_hbm, acc.at[pl.ds(s * NB, NB)])
        if stage_table:
            table = maybe_tab[0]
            pltpu.sync_copy(table_hbm.at[pl.ds(s * NB, NB)],
                            table.at[pl.ds(s * NB, NB)])
        else:
            table = table_hbm
        plsc.subcore_barrier()
        ebase0 = wid * EPW

        def body(i, carry):
            ebase = ebase0 + i * chunk
            pltpu.sync_copy(src_hbm.at[pl.ds(ebase, chunk)], sidx)
            pltpu.sync_copy(dst_hbm.at[pl.ds(ebase, chunk)], didx)
            for j in range(kc):
                pltpu.async_copy(table.at[sidx.at[pl.ds(j * LANE, LANE)]],
                                 rows.at[pl.ds(j * LANE, LANE)], sem)
            for j in range(kc):
                pltpu.make_async_copy(table.at[sidx.at[pl.ds(j * LANE, LANE)]],
                                      rows.at[pl.ds(j * LANE, LANE)],
                                      sem).wait()
            for j in range(kc):
                pltpu.sync_copy(rows.at[pl.ds(j * LANE, LANE)],
                                acc.at[didx.at[pl.ds(j * LANE, LANE)]],
                                add=True)
            return carry

        lax.fori_loop(0, nchunks, body, 0)
        plsc.subcore_barrier()
        pltpu.sync_copy(acc.at[pl.ds(s * NB, NB)],
                        out_hbm.at[c].at[pl.ds(s * NB, NB)])

    return prop_kernel


_deg_call = _make_deg()
_prop8_call = _make_prop(8, stage_table=True, kc=12)
_prop16_call = _make_prop(16, stage_table=False, kc=8)

BM = 2048
GRID = pl.cdiv(N, BM)


def _prep_body(degp_ref, x_ref, dinv_ref, xs_ref):
    deg = degp_ref[0] + degp_ref[1] + 1.0
    dinv = lax.rsqrt(deg)
    dinv_ref[...] = dinv
    xsc = x_ref[...] * dinv
    xs_ref[...] = jnp.concatenate([xsc, jnp.zeros_like(xsc)], axis=1)


def _mid_body(zp_ref, xs_ref, dinv_ref, w1_ref, b1_ref, w2_ref, gs_ref):
    dinv = dinv_ref[...]
    z = (zp_ref[0] + zp_ref[1] + xs_ref[...])[:, 0:4] * dinv
    h = jnp.dot(z, w1_ref[...], preferred_element_type=jnp.float32,
                precision=lax.Precision.HIGHEST)
    h = jnp.maximum(h + b1_ref[...], 0.0)
    g = jnp.dot(h, w2_ref[...], preferred_element_type=jnp.float32,
                precision=lax.Precision.HIGHEST)
    gs_ref[...] = g * dinv


def _post_body(qp_ref, gs_ref, dinv_ref, b2_ref, out_ref):
    out_ref[...] = ((qp_ref[0] + qp_ref[1] + gs_ref[...]) * dinv_ref[...]
                    + b2_ref[...])


def _prep_call(degp3, x):
    return pl.pallas_call(
        _prep_body,
        grid=(GRID,),
        in_specs=[pl.BlockSpec((NC, BM, 1), lambda i: (0, i, 0)),
                  pl.BlockSpec((BM, 4), lambda i: (i, 0))],
        out_specs=[pl.BlockSpec((BM, 1), lambda i: (i, 0)),
                   pl.BlockSpec((BM, 8), lambda i: (i, 0))],
        out_shape=[jax.ShapeDtypeStruct((N, 1), jnp.float32),
                   jax.ShapeDtypeStruct((NPAD, 8), jnp.float32)],
    )(degp3, x)


def _mid_call(zp, xs, dinv, w1, b1, w2):
    return pl.pallas_call(
        _mid_body,
        grid=(GRID,),
        in_specs=[pl.BlockSpec((NC, BM, 8), lambda i: (0, i, 0)),
                  pl.BlockSpec((BM, 8), lambda i: (i, 0)),
                  pl.BlockSpec((BM, 1), lambda i: (i, 0)),
                  pl.BlockSpec((4, 32), lambda i: (0, 0)),
                  pl.BlockSpec((1, 32), lambda i: (0, 0)),
                  pl.BlockSpec((32, 16), lambda i: (0, 0))],
        out_specs=pl.BlockSpec((BM, 16), lambda i: (i, 0)),
        out_shape=jax.ShapeDtypeStruct((NPAD, 16), jnp.float32),
    )(zp, xs, dinv, w1, b1, w2)


def _post_call(qp, gs, dinv, b2):
    return pl.pallas_call(
        _post_body,
        grid=(GRID,),
        in_specs=[pl.BlockSpec((NC, BM, 16), lambda i: (0, i, 0)),
                  pl.BlockSpec((BM, 16), lambda i: (i, 0)),
                  pl.BlockSpec((BM, 1), lambda i: (i, 0)),
                  pl.BlockSpec((1, 16), lambda i: (0, 0))],
        out_specs=pl.BlockSpec((BM, 16), lambda i: (i, 0)),
        out_shape=jax.ShapeDtypeStruct((N, 16), jnp.float32),
    )(qp, gs, dinv, b2)


def kernel(x, edge_index, W1, b1, W2, b2):
    src = edge_index[0].astype(jnp.int32)
    dst = edge_index[1].astype(jnp.int32)
    pad = jnp.full((EP - E,), N, jnp.int32)
    src2 = jnp.concatenate([src, pad])
    dst2 = jnp.concatenate([dst, pad])
    ones = jnp.ones((LANE,), jnp.float32)

    degp = _deg_call(dst2, ones, jnp.zeros((NB,), jnp.float32))
    dinv, xs = _prep_call(degp.reshape(NC, NPAD, 1), x)
    zp = _prop8_call(src2, dst2, xs, jnp.zeros((NB, 8), jnp.float32))
    gs = _mid_call(zp, xs, dinv, W1, b1.reshape(1, 32), W2)
    qp = _prop16_call(src2, dst2, gs, jnp.zeros((NB, 16), jnp.float32))
    return _post_call(qp, gs, dinv, b2.reshape(1, 16))


# speedup vs baseline: 53.8121x; 1.0662x over previous
import functools

import jax
import jax.numpy as jnp
from jax import lax
from jax.experimental import pallas as pl
from jax.experimental.pallas import tpu as pltpu
from jax.experimental.pallas import tpu_sc as plsc

N = 100000
E = 3200000
NC, NS = 2, 16
NW = NC * NS
LANE = 128
KC = 24
CHUNK = KC * LANE
EPW = ((E + NW * CHUNK - 1) // (NW * CHUNK)) * CHUNK
EP = EPW * NW
NCHUNKS = EPW // CHUNK
NB = 6272
NPAD = NB * NS

_MESH = plsc.VectorSubcoreMesh(core_axis_name="c", subcore_axis_name="s")


def _make_deg(kc):
    chunk = kc * LANE
    nchunks = EPW // chunk
    assert nchunks * chunk == EPW and nchunks % 2 == 0

    @functools.partial(
        pl.kernel,
        out_type=jax.ShapeDtypeStruct((NC, NPAD), jnp.float32),
        mesh=_MESH,
        scratch_types=[
            pltpu.VMEM((chunk,), jnp.int32),
            pltpu.VMEM((chunk,), jnp.int32),
            pltpu.VMEM((LANE,), jnp.float32),
            pltpu.VMEM_SHARED((NPAD,), jnp.float32),
            pltpu.SemaphoreType.DMA,
            pltpu.SemaphoreType.DMA,
        ],
        compiler_params=pltpu.CompilerParams(use_tc_tiling_on_sc=False),
    )
    def deg_kernel(dst_hbm, ones_hbm, zeros_hbm, out_hbm,
                   didx0, didx1, ones_v, acc, ss0, ss1):
        c = lax.axis_index("c")
        s = lax.axis_index("s")
        wid = c * NS + s
        didxs = (didx0, didx1)
        sss = (ss0, ss1)
        pltpu.sync_copy(ones_hbm, ones_v)
        pltpu.sync_copy(zeros_hbm, acc.at[pl.ds(s * NB, NB)])
        plsc.subcore_barrier()
        ebase0 = wid * EPW

        def scatter_descs(b):
            return [pltpu.make_async_copy(
                ones_v, acc.at[didxs[b].at[pl.ds(j * LANE, LANE)]], sss[b])
                for j in range(kc)]

        def body(i, carry):
            for b in range(2):
                @pl.when(i > 0)
                def _():
                    for d in scatter_descs(b):
                        d.wait()

                ebase = ebase0 + (2 * i + b) * chunk
                pltpu.sync_copy(dst_hbm.at[pl.ds(ebase, chunk)], didxs[b])
                for j in range(kc):
                    pltpu.async_copy(
                        ones_v, acc.at[didxs[b].at[pl.ds(j * LANE, LANE)]],
                        sss[b], add=True)
            return carry

        lax.fori_loop(0, nchunks // 2, body, 0)
        for b in range(2):
            for d in scatter_descs(b):
                d.wait()
        plsc.subcore_barrier()
        pltpu.sync_copy(acc.at[pl.ds(s * NB, NB)],
                        out_hbm.at[c].at[pl.ds(s * NB, NB)])

    return deg_kernel


def _make_prop(C, stage_table, kc):
    chunk = kc * LANE
    nchunks = EPW // chunk
    assert nchunks * chunk == EPW and nchunks % 2 == 0
    scratch = [
        pltpu.VMEM((chunk,), jnp.int32),
        pltpu.VMEM((chunk,), jnp.int32),
        pltpu.VMEM((chunk,), jnp.int32),
        pltpu.VMEM((chunk,), jnp.int32),
        pltpu.VMEM((chunk, C), jnp.float32),
        pltpu.VMEM((chunk, C), jnp.float32),
        pltpu.VMEM_SHARED((NPAD, C), jnp.float32),
        pltpu.SemaphoreType.DMA,
        pltpu.SemaphoreType.DMA,
        pltpu.SemaphoreType.DMA,
        pltpu.SemaphoreType.DMA,
    ]
    if stage_table:
        scratch.append(pltpu.VMEM_SHARED((NPAD, C), jnp.float32))

    @functools.partial(
        pl.kernel,
        out_type=jax.ShapeDtypeStruct((NC, NPAD, C), jnp.float32),
        mesh=_MESH,
        scratch_types=scratch,
        compiler_params=pltpu.CompilerParams(use_tc_tiling_on_sc=False),
    )
    def prop_kernel(src_hbm, dst_hbm, table_hbm, zeros_hbm, out_hbm,
                    sidx0, sidx1, didx0, didx1, rows0, rows1, acc,
                    sg0, sg1, ss0, ss1, *maybe_tab):
        c = lax.axis_index("c")
        s = lax.axis_index("s")
        wid = c * NS + s
        sidxs, didxs = (sidx0, sidx1), (didx0, didx1)
        rowss, sgs, sss = (rows0, rows1), (sg0, sg1), (ss0, ss1)
        pltpu.sync_copy(zeros_hbm, acc.at[pl.ds(s * NB, NB)])
        if stage_table:
            table = maybe_tab[0]
            pltpu.sync_copy(table_hbm.at[pl.ds(s * NB, NB)],
                            table.at[pl.ds(s * NB, NB)])
        else:
            table = table_hbm
        plsc.subcore_barrier()
        ebase0 = wid * EPW

        def scatter_descs(b):
            return [pltpu.make_async_copy(
                rowss[b].at[pl.ds(j * LANE, LANE)],
                acc.at[didxs[b].at[pl.ds(j * LANE, LANE)]], sss[b])
                for j in range(kc)]

        def body(i, carry):
            for b in range(2):
                @pl.when(i > 0)
                def _():
                    for d in scatter_descs(b):
                        d.wait()

                ebase = ebase0 + (2 * i + b) * chunk
                pltpu.sync_copy(src_hbm.at[pl.ds(ebase, chunk)], sidxs[b])
                pltpu.sync_copy(dst_hbm.at[pl.ds(ebase, chunk)], didxs[b])
                for j in range(kc):
                    pltpu.async_copy(
                        table.at[sidxs[b].at[pl.ds(j * LANE, LANE)]],
                        rowss[b].at[pl.ds(j * LANE, LANE)], sgs[b])
                for j in range(kc):
                    pltpu.make_async_copy(
                        table.at[sidxs[b].at[pl.ds(j * LANE, LANE)]],
                        rowss[b].at[pl.ds(j * LANE, LANE)], sgs[b]).wait()
                for j in range(kc):
                    pltpu.async_copy(
                        rowss[b].at[pl.ds(j * LANE, LANE)],
                        acc.at[didxs[b].at[pl.ds(j * LANE, LANE)]],
                        sss[b], add=True)
            return carry

        lax.fori_loop(0, nchunks // 2, body, 0)
        for b in range(2):
            for d in scatter_descs(b):
                d.wait()
        plsc.subcore_barrier()
        pltpu.sync_copy(acc.at[pl.ds(s * NB, NB)],
                        out_hbm.at[c].at[pl.ds(s * NB, NB)])

    return prop_kernel


_deg_call = _make_deg(kc=12)
_prop8_call = _make_prop(8, stage_table=True, kc=11)
_prop16_call = _make_prop(16, stage_table=False, kc=6)

BM = 2048
GRID = pl.cdiv(N, BM)


def _prep_body(degp_ref, x_ref, dinv_ref, xs_ref):
    deg = degp_ref[0] + degp_ref[1] + 1.0
    dinv = lax.rsqrt(deg)
    dinv_ref[...] = dinv
    xsc = x_ref[...] * dinv
    xs_ref[...] = jnp.concatenate([xsc, jnp.zeros_like(xsc)], axis=1)


def _mid_body(zp_ref, xs_ref, dinv_ref, w1_ref, b1_ref, w2_ref, gs_ref):
    dinv = dinv_ref[...]
    z = (zp_ref[0] + zp_ref[1] + xs_ref[...])[:, 0:4] * dinv
    h = jnp.dot(z, w1_ref[...], preferred_element_type=jnp.float32,
                precision=lax.Precision.HIGHEST)
    h = jnp.maximum(h + b1_ref[...], 0.0)
    g = jnp.dot(h, w2_ref[...], preferred_element_type=jnp.float32,
                precision=lax.Precision.HIGHEST)
    gs_ref[...] = g * dinv


def _post_body(qp_ref, gs_ref, dinv_ref, b2_ref, out_ref):
    out_ref[...] = ((qp_ref[0] + qp_ref[1] + gs_ref[...]) * dinv_ref[...]
                    + b2_ref[...])


def _prep_call(degp3, x):
    return pl.pallas_call(
        _prep_body,
        grid=(GRID,),
        in_specs=[pl.BlockSpec((NC, BM, 1), lambda i: (0, i, 0)),
                  pl.BlockSpec((BM, 4), lambda i: (i, 0))],
        out_specs=[pl.BlockSpec((BM, 1), lambda i: (i, 0)),
                   pl.BlockSpec((BM, 8), lambda i: (i, 0))],
        out_shape=[jax.ShapeDtypeStruct((N, 1), jnp.float32),
                   jax.ShapeDtypeStruct((NPAD, 8), jnp.float32)],
    )(degp3, x)


def _mid_call(zp, xs, dinv, w1, b1, w2):
    return pl.pallas_call(
        _mid_body,
        grid=(GRID,),
        in_specs=[pl.BlockSpec((NC, BM, 8), lambda i: (0, i, 0)),
                  pl.BlockSpec((BM, 8), lambda i: (i, 0)),
                  pl.BlockSpec((BM, 1), lambda i: (i, 0)),
                  pl.BlockSpec((4, 32), lambda i: (0, 0)),
                  pl.BlockSpec((1, 32), lambda i: (0, 0)),
                  pl.BlockSpec((32, 16), lambda i: (0, 0))],
        out_specs=pl.BlockSpec((BM, 16), lambda i: (i, 0)),
        out_shape=jax.ShapeDtypeStruct((NPAD, 16), jnp.float32),
    )(zp, xs, dinv, w1, b1, w2)


def _post_call(qp, gs, dinv, b2):
    return pl.pallas_call(
        _post_body,
        grid=(GRID,),
        in_specs=[pl.BlockSpec((NC, BM, 16), lambda i: (0, i, 0)),
                  pl.BlockSpec((BM, 16), lambda i: (i, 0)),
                  pl.BlockSpec((BM, 1), lambda i: (i, 0)),
                  pl.BlockSpec((1, 16), lambda i: (0, 0))],
        out_specs=pl.BlockSpec((BM, 16), lambda i: (i, 0)),
        out_shape=jax.ShapeDtypeStruct((N, 16), jnp.float32),
    )(qp, gs, dinv, b2)


def kernel(x, edge_index, W1, b1, W2, b2):
    src = edge_index[0].astype(jnp.int32)
    dst = edge_index[1].astype(jnp.int32)
    pad = jnp.full((EP - E,), N, jnp.int32)
    src2 = jnp.concatenate([src, pad])
    dst2 = jnp.concatenate([dst, pad])
    ones = jnp.ones((LANE,), jnp.float32)

    degp = _deg_call(dst2, ones, jnp.zeros((NB,), jnp.float32))
    dinv, xs = _prep_call(degp.reshape(NC, NPAD, 1), x)
    zp = _prop8_call(src2, dst2, xs, jnp.zeros((NB, 8), jnp.float32))
    gs = _mid_call(zp, xs, dinv, W1, b1.reshape(1, 32), W2)
    qp = _prop16_call(src2, dst2, gs, jnp.zeros((NB, 16), jnp.float32))
    return _post_call(qp, gs, dinv, b2.reshape(1, 16))


# speedup vs baseline: 54.0601x; 1.0046x over previous
import functools

import jax
import jax.numpy as jnp
from jax import lax
from jax.experimental import pallas as pl
from jax.experimental.pallas import tpu as pltpu
from jax.experimental.pallas import tpu_sc as plsc

N = 100000
E = 3200000
NC, NS = 2, 16
NW = NC * NS
LANE = 128
KC = 24
CHUNK = KC * LANE
EPW = ((E + NW * CHUNK - 1) // (NW * CHUNK)) * CHUNK
EP = EPW * NW
NCHUNKS = EPW // CHUNK
NB = 6272
NPAD = NB * NS

_MESH = plsc.VectorSubcoreMesh(core_axis_name="c", subcore_axis_name="s")


def _make_deg(kc, sl=LANE):
    chunk = kc * sl
    nchunks = EPW // chunk
    assert nchunks * chunk == EPW and nchunks % 2 == 0

    @functools.partial(
        pl.kernel,
        out_type=jax.ShapeDtypeStruct((NC, NPAD), jnp.float32),
        mesh=_MESH,
        scratch_types=[
            pltpu.VMEM((chunk,), jnp.int32),
            pltpu.VMEM((chunk,), jnp.int32),
            pltpu.VMEM((sl,), jnp.float32),
            pltpu.VMEM_SHARED((NPAD,), jnp.float32),
            pltpu.SemaphoreType.DMA,
            pltpu.SemaphoreType.DMA,
        ],
        compiler_params=pltpu.CompilerParams(use_tc_tiling_on_sc=False),
    )
    def deg_kernel(dst_hbm, ones_hbm, zeros_hbm, out_hbm,
                   didx0, didx1, ones_v, acc, ss0, ss1):
        c = lax.axis_index("c")
        s = lax.axis_index("s")
        wid = c * NS + s
        didxs = (didx0, didx1)
        sss = (ss0, ss1)
        pltpu.sync_copy(ones_hbm, ones_v)
        pltpu.sync_copy(zeros_hbm, acc.at[pl.ds(s * NB, NB)])
        plsc.subcore_barrier()
        ebase0 = wid * EPW

        def scatter_descs(b):
            return [pltpu.make_async_copy(
                ones_v, acc.at[didxs[b].at[pl.ds(j * sl, sl)]], sss[b])
                for j in range(kc)]

        def body(i, carry):
            for b in range(2):
                @pl.when(i > 0)
                def _():
                    for d in scatter_descs(b):
                        d.wait()

                ebase = ebase0 + (2 * i + b) * chunk
                pltpu.sync_copy(dst_hbm.at[pl.ds(ebase, chunk)], didxs[b])
                for j in range(kc):
                    pltpu.async_copy(
                        ones_v, acc.at[didxs[b].at[pl.ds(j * sl, sl)]],
                        sss[b], add=True)
            return carry

        lax.fori_loop(0, nchunks // 2, body, 0)
        for b in range(2):
            for d in scatter_descs(b):
                d.wait()
        plsc.subcore_barrier()
        pltpu.sync_copy(acc.at[pl.ds(s * NB, NB)],
                        out_hbm.at[c].at[pl.ds(s * NB, NB)])

    return deg_kernel


def _make_prop(C, stage_table, kc, sl=LANE):
    chunk = kc * sl
    nchunks = EPW // chunk
    assert nchunks * chunk == EPW and nchunks % 2 == 0
    scratch = [
        pltpu.VMEM((chunk,), jnp.int32),
        pltpu.VMEM((chunk,), jnp.int32),
        pltpu.VMEM((chunk,), jnp.int32),
        pltpu.VMEM((chunk,), jnp.int32),
        pltpu.VMEM((chunk, C), jnp.float32),
        pltpu.VMEM((chunk, C), jnp.float32),
        pltpu.VMEM_SHARED((NPAD, C), jnp.float32),
        pltpu.SemaphoreType.DMA,
        pltpu.SemaphoreType.DMA,
        pltpu.SemaphoreType.DMA,
        pltpu.SemaphoreType.DMA,
    ]
    if stage_table:
        scratch.append(pltpu.VMEM_SHARED((NPAD, C), jnp.float32))

    @functools.partial(
        pl.kernel,
        out_type=jax.ShapeDtypeStruct((NC, NPAD, C), jnp.float32),
        mesh=_MESH,
        scratch_types=scratch,
        compiler_params=pltpu.CompilerParams(use_tc_tiling_on_sc=False),
    )
    def prop_kernel(src_hbm, dst_hbm, table_hbm, zeros_hbm, out_hbm,
                    sidx0, sidx1, didx0, didx1, rows0, rows1, acc,
                    sg0, sg1, ss0, ss1, *maybe_tab):
        c = lax.axis_index("c")
        s = lax.axis_index("s")
        wid = c * NS + s
        sidxs, didxs = (sidx0, sidx1), (didx0, didx1)
        rowss, sgs, sss = (rows0, rows1), (sg0, sg1), (ss0, ss1)
        pltpu.sync_copy(zeros_hbm, acc.at[pl.ds(s * NB, NB)])
        if stage_table:
            table = maybe_tab[0]
            pltpu.sync_copy(table_hbm.at[pl.ds(s * NB, NB)],
                            table.at[pl.ds(s * NB, NB)])
        else:
            table = table_hbm
        plsc.subcore_barrier()
        ebase0 = wid * EPW

        def scatter_descs(b):
            return [pltpu.make_async_copy(
                rowss[b].at[pl.ds(j * sl, sl)],
                acc.at[didxs[b].at[pl.ds(j * sl, sl)]], sss[b])
                for j in range(kc)]

        def body(i, carry):
            for b in range(2):
                @pl.when(i > 0)
                def _():
                    for d in scatter_descs(b):
                        d.wait()

                ebase = ebase0 + (2 * i + b) * chunk
                pltpu.sync_copy(src_hbm.at[pl.ds(ebase, chunk)], sidxs[b])
                pltpu.sync_copy(dst_hbm.at[pl.ds(ebase, chunk)], didxs[b])
                for j in range(kc):
                    pltpu.async_copy(
                        table.at[sidxs[b].at[pl.ds(j * sl, sl)]],
                        rowss[b].at[pl.ds(j * sl, sl)], sgs[b])
                for j in range(kc):
                    pltpu.make_async_copy(
                        table.at[sidxs[b].at[pl.ds(j * sl, sl)]],
                        rowss[b].at[pl.ds(j * sl, sl)], sgs[b]).wait()
                for j in range(kc):
                    pltpu.async_copy(
                        rowss[b].at[pl.ds(j * sl, sl)],
                        acc.at[didxs[b].at[pl.ds(j * sl, sl)]],
                        sss[b], add=True)
            return carry

        lax.fori_loop(0, nchunks // 2, body, 0)
        for b in range(2):
            for d in scatter_descs(b):
                d.wait()
        plsc.subcore_barrier()
        pltpu.sync_copy(acc.at[pl.ds(s * NB, NB)],
                        out_hbm.at[c].at[pl.ds(s * NB, NB)])

    return prop_kernel


_deg_call = _make_deg(kc=1, sl=1536)
_prop8_call = _make_prop(8, stage_table=True, kc=1, sl=1408)
_prop16_call = _make_prop(16, stage_table=False, kc=1, sl=768)

BM = 2048
GRID = pl.cdiv(N, BM)


def _prep_body(degp_ref, x_ref, dinv_ref, xs_ref):
    deg = degp_ref[0] + degp_ref[1] + 1.0
    dinv = lax.rsqrt(deg)
    dinv_ref[...] = dinv
    xsc = x_ref[...] * dinv
    xs_ref[...] = jnp.concatenate([xsc, jnp.zeros_like(xsc)], axis=1)


def _mid_body(zp_ref, xs_ref, dinv_ref, w1_ref, b1_ref, w2_ref, gs_ref):
    dinv = dinv_ref[...]
    z = (zp_ref[0] + zp_ref[1] + xs_ref[...])[:, 0:4] * dinv
    h = jnp.dot(z, w1_ref[...], preferred_element_type=jnp.float32,
                precision=lax.Precision.HIGHEST)
    h = jnp.maximum(h + b1_ref[...], 0.0)
    g = jnp.dot(h, w2_ref[...], preferred_element_type=jnp.float32,
                precision=lax.Precision.HIGHEST)
    gs_ref[...] = g * dinv


def _post_body(qp_ref, gs_ref, dinv_ref, b2_ref, out_ref):
    out_ref[...] = ((qp_ref[0] + qp_ref[1] + gs_ref[...]) * dinv_ref[...]
                    + b2_ref[...])


def _prep_call(degp3, x):
    return pl.pallas_call(
        _prep_body,
        grid=(GRID,),
        in_specs=[pl.BlockSpec((NC, BM, 1), lambda i: (0, i, 0)),
                  pl.BlockSpec((BM, 4), lambda i: (i, 0))],
        out_specs=[pl.BlockSpec((BM, 1), lambda i: (i, 0)),
                   pl.BlockSpec((BM, 8), lambda i: (i, 0))],
        out_shape=[jax.ShapeDtypeStruct((N, 1), jnp.float32),
                   jax.ShapeDtypeStruct((NPAD, 8), jnp.float32)],
    )(degp3, x)


def _mid_call(zp, xs, dinv, w1, b1, w2):
    return pl.pallas_call(
        _mid_body,
        grid=(GRID,),
        in_specs=[pl.BlockSpec((NC, BM, 8), lambda i: (0, i, 0)),
                  pl.BlockSpec((BM, 8), lambda i: (i, 0)),
                  pl.BlockSpec((BM, 1), lambda i: (i, 0)),
                  pl.BlockSpec((4, 32), lambda i: (0, 0)),
                  pl.BlockSpec((1, 32), lambda i: (0, 0)),
                  pl.BlockSpec((32, 16), lambda i: (0, 0))],
        out_specs=pl.BlockSpec((BM, 16), lambda i: (i, 0)),
        out_shape=jax.ShapeDtypeStruct((NPAD, 16), jnp.float32),
    )(zp, xs, dinv, w1, b1, w2)


def _post_call(qp, gs, dinv, b2):
    return pl.pallas_call(
        _post_body,
        grid=(GRID,),
        in_specs=[pl.BlockSpec((NC, BM, 16), lambda i: (0, i, 0)),
                  pl.BlockSpec((BM, 16), lambda i: (i, 0)),
                  pl.BlockSpec((BM, 1), lambda i: (i, 0)),
                  pl.BlockSpec((1, 16), lambda i: (0, 0))],
        out_specs=pl.BlockSpec((BM, 16), lambda i: (i, 0)),
        out_shape=jax.ShapeDtypeStruct((N, 16), jnp.float32),
    )(qp, gs, dinv, b2)


def kernel(x, edge_index, W1, b1, W2, b2):
    src = edge_index[0].astype(jnp.int32)
    dst = edge_index[1].astype(jnp.int32)
    pad = jnp.full((EP - E,), N, jnp.int32)
    src2 = jnp.concatenate([src, pad])
    dst2 = jnp.concatenate([dst, pad])
    ones = jnp.ones((1536,), jnp.float32)

    degp = _deg_call(dst2, ones, jnp.zeros((NB,), jnp.float32))
    dinv, xs = _prep_call(degp.reshape(NC, NPAD, 1), x)
    zp = _prop8_call(src2, dst2, xs, jnp.zeros((NB, 8), jnp.float32))
    gs = _mid_call(zp, xs, dinv, W1, b1.reshape(1, 32), W2)
    qp = _prop16_call(src2, dst2, gs, jnp.zeros((NB, 16), jnp.float32))
    return _post_call(qp, gs, dinv, b2.reshape(1, 16))


# speedup vs baseline: 62.6646x; 1.1592x over previous
import functools

import jax
import jax.numpy as jnp
from jax import lax
from jax.experimental import pallas as pl
from jax.experimental.pallas import tpu as pltpu
from jax.experimental.pallas import tpu_sc as plsc

N = 100000
E = 3200000
NC, NS = 2, 16
NW = NC * NS
LANE = 128
KC = 24
CHUNK = KC * LANE
EPW = ((E + NW * CHUNK - 1) // (NW * CHUNK)) * CHUNK
EP = EPW * NW
NCHUNKS = EPW // CHUNK
NB = 6272
NPAD = NB * NS

_MESH = plsc.VectorSubcoreMesh(core_axis_name="c", subcore_axis_name="s")


def _make_deg(kc, sl=LANE):
    chunk = kc * sl
    nchunks = EPW // chunk
    assert nchunks * chunk == EPW and nchunks % 2 == 0

    @functools.partial(
        pl.kernel,
        out_type=jax.ShapeDtypeStruct((NC, NPAD), jnp.float32),
        mesh=_MESH,
        scratch_types=[
            pltpu.VMEM((chunk,), jnp.int32),
            pltpu.VMEM((chunk,), jnp.int32),
            pltpu.VMEM((sl,), jnp.float32),
            pltpu.VMEM_SHARED((NPAD,), jnp.float32),
            pltpu.SemaphoreType.DMA,
            pltpu.SemaphoreType.DMA,
        ],
        compiler_params=pltpu.CompilerParams(use_tc_tiling_on_sc=False),
    )
    def deg_kernel(dst_hbm, ones_hbm, zeros_hbm, out_hbm,
                   didx0, didx1, ones_v, acc, ss0, ss1):
        c = lax.axis_index("c")
        s = lax.axis_index("s")
        wid = c * NS + s
        didxs = (didx0, didx1)
        sss = (ss0, ss1)
        pltpu.sync_copy(ones_hbm, ones_v)
        pltpu.sync_copy(zeros_hbm, acc.at[pl.ds(s * NB, NB)])
        plsc.subcore_barrier()
        ebase0 = wid * EPW

        def scatter_descs(b):
            return [pltpu.make_async_copy(
                ones_v, acc.at[didxs[b].at[pl.ds(j * sl, sl)]], sss[b])
                for j in range(kc)]

        def body(i, carry):
            for b in range(2):
                @pl.when(i > 0)
                def _():
                    for d in scatter_descs(b):
                        d.wait()

                ebase = ebase0 + (2 * i + b) * chunk
                pltpu.sync_copy(dst_hbm.at[pl.ds(ebase, chunk)], didxs[b])
                for j in range(kc):
                    pltpu.async_copy(
                        ones_v, acc.at[didxs[b].at[pl.ds(j * sl, sl)]],
                        sss[b], add=True)
            return carry

        lax.fori_loop(0, nchunks // 2, body, 0)
        for b in range(2):
            for d in scatter_descs(b):
                d.wait()
        plsc.subcore_barrier()
        pltpu.sync_copy(acc.at[pl.ds(s * NB, NB)],
                        out_hbm.at[c].at[pl.ds(s * NB, NB)])

    return deg_kernel


def _make_prop(C, stage_table, kc, sl=LANE):
    chunk = kc * sl
    nchunks = EPW // chunk
    assert nchunks * chunk == EPW and nchunks % 2 == 0
    scratch = [
        pltpu.VMEM((chunk,), jnp.int32),
        pltpu.VMEM((chunk,), jnp.int32),
        pltpu.VMEM((chunk,), jnp.int32),
        pltpu.VMEM((chunk,), jnp.int32),
        pltpu.VMEM((chunk, C), jnp.float32),
        pltpu.VMEM((chunk, C), jnp.float32),
        pltpu.VMEM_SHARED((NPAD, C), jnp.float32),
        pltpu.SemaphoreType.DMA,
        pltpu.SemaphoreType.DMA,
        pltpu.SemaphoreType.DMA,
        pltpu.SemaphoreType.DMA,
    ]
    if stage_table:
        scratch.append(pltpu.VMEM_SHARED((NPAD, C), jnp.float32))

    @functools.partial(
        pl.kernel,
        out_type=jax.ShapeDtypeStruct((NC, NPAD, C), jnp.float32),
        mesh=_MESH,
        scratch_types=scratch,
        compiler_params=pltpu.CompilerParams(use_tc_tiling_on_sc=False),
    )
    def prop_kernel(src_hbm, dst_hbm, table_hbm, zeros_hbm, out_hbm,
                    sidx0, sidx1, didx0, didx1, rows0, rows1, acc,
                    sg0, sg1, ss0, ss1, *maybe_tab):
        c = lax.axis_index("c")
        s = lax.axis_index("s")
        wid = c * NS + s
        sidxs, didxs = (sidx0, sidx1), (didx0, didx1)
        rowss, sgs, sss = (rows0, rows1), (sg0, sg1), (ss0, ss1)
        pltpu.sync_copy(zeros_hbm, acc.at[pl.ds(s * NB, NB)])
        if stage_table:
            table = maybe_tab[0]
            pltpu.sync_copy(table_hbm.at[pl.ds(s * NB, NB)],
                            table.at[pl.ds(s * NB, NB)])
        else:
            table = table_hbm
        plsc.subcore_barrier()
        ebase0 = wid * EPW

        def scatter_descs(b):
            return [pltpu.make_async_copy(
                rowss[b].at[pl.ds(j * sl, sl)],
                acc.at[didxs[b].at[pl.ds(j * sl, sl)]], sss[b])
                for j in range(kc)]

        def body(i, carry):
            for b in range(2):
                @pl.when(i > 0)
                def _():
                    for d in scatter_descs(b):
                        d.wait()

                ebase = ebase0 + (2 * i + b) * chunk
                pltpu.sync_copy(src_hbm.at[pl.ds(ebase, chunk)], sidxs[b])
                pltpu.sync_copy(dst_hbm.at[pl.ds(ebase, chunk)], didxs[b])
                for j in range(kc):
                    pltpu.async_copy(
                        table.at[sidxs[b].at[pl.ds(j * sl, sl)]],
                        rowss[b].at[pl.ds(j * sl, sl)], sgs[b])
                for j in range(kc):
                    pltpu.make_async_copy(
                        table.at[sidxs[b].at[pl.ds(j * sl, sl)]],
                        rowss[b].at[pl.ds(j * sl, sl)], sgs[b]).wait()
                for j in range(kc):
                    pltpu.async_copy(
                        rowss[b].at[pl.ds(j * sl, sl)],
                        acc.at[didxs[b].at[pl.ds(j * sl, sl)]],
                        sss[b], add=True)
            return carry

        lax.fori_loop(0, nchunks // 2, body, 0)
        for b in range(2):
            for d in scatter_descs(b):
                d.wait()
        plsc.subcore_barrier()
        pltpu.sync_copy(acc.at[pl.ds(s * NB, NB)],
                        out_hbm.at[c].at[pl.ds(s * NB, NB)])

    return prop_kernel


def _make_prop_cols(kc, sl):
    chunk = kc * sl
    epc = EP // NS
    nchunks = epc // chunk
    assert nchunks * chunk == epc and nchunks % 2 == 0

    @functools.partial(
        pl.kernel,
        out_type=jax.ShapeDtypeStruct((NC, NPAD, 8), jnp.float32),
        mesh=_MESH,
        scratch_types=[
            pltpu.VMEM((chunk,), jnp.int32),
            pltpu.VMEM((chunk,), jnp.int32),
            pltpu.VMEM((chunk,), jnp.int32),
            pltpu.VMEM((chunk,), jnp.int32),
            pltpu.VMEM((chunk, 8), jnp.float32),
            pltpu.VMEM((chunk, 8), jnp.float32),
            pltpu.VMEM_SHARED((NPAD, 8), jnp.float32),
            pltpu.VMEM_SHARED((NPAD, 8), jnp.float32),
            pltpu.SemaphoreType.DMA,
            pltpu.SemaphoreType.DMA,
            pltpu.SemaphoreType.DMA,
            pltpu.SemaphoreType.DMA,
        ],
        compiler_params=pltpu.CompilerParams(use_tc_tiling_on_sc=False),
    )
    def prop_kernel(src_hbm, dst_hbm, table_hbm, zeros_hbm, out_hbm,
                    sidx0, sidx1, didx0, didx1, rows0, rows1, acc, table,
                    sg0, sg1, ss0, ss1):
        c = lax.axis_index("c")
        s = lax.axis_index("s")
        sidxs, didxs = (sidx0, sidx1), (didx0, didx1)
        rowss, sgs, sss = (rows0, rows1), (sg0, sg1), (ss0, ss1)
        pltpu.sync_copy(zeros_hbm, acc.at[pl.ds(s * NB, NB)])
        pltpu.sync_copy(table_hbm.at[c].at[pl.ds(s * NB, NB)],
                        table.at[pl.ds(s * NB, NB)])
        plsc.subcore_barrier()
        ebase0 = s * epc

        def scatter_descs(b):
            return [pltpu.make_async_copy(
                rowss[b].at[pl.ds(j * sl, sl)],
                acc.at[didxs[b].at[pl.ds(j * sl, sl)]], sss[b])
                for j in range(kc)]

        def body(i, carry):
            for b in range(2):
                @pl.when(i > 0)
                def _():
                    for d in scatter_descs(b):
                        d.wait()

                ebase = ebase0 + (2 * i + b) * chunk
                pltpu.sync_copy(src_hbm.at[pl.ds(ebase, chunk)], sidxs[b])
                pltpu.sync_copy(dst_hbm.at[pl.ds(ebase, chunk)], didxs[b])
                for j in range(kc):
                    pltpu.async_copy(
                        table.at[sidxs[b].at[pl.ds(j * sl, sl)]],
                        rowss[b].at[pl.ds(j * sl, sl)], sgs[b])
                for j in range(kc):
                    pltpu.make_async_copy(
                        table.at[sidxs[b].at[pl.ds(j * sl, sl)]],
                        rowss[b].at[pl.ds(j * sl, sl)], sgs[b]).wait()
                for j in range(kc):
                    pltpu.async_copy(
                        rowss[b].at[pl.ds(j * sl, sl)],
                        acc.at[didxs[b].at[pl.ds(j * sl, sl)]],
                        sss[b], add=True)
            return carry

        lax.fori_loop(0, nchunks // 2, body, 0)
        for b in range(2):
            for d in scatter_descs(b):
                d.wait()
        plsc.subcore_barrier()
        pltpu.sync_copy(acc.at[pl.ds(s * NB, NB)],
                        out_hbm.at[c].at[pl.ds(s * NB, NB)])

    return prop_kernel


_deg_call = _make_deg(kc=1, sl=1536)
_prop8_call = _make_prop(8, stage_table=True, kc=1, sl=1408)
_prop2_call = _make_prop_cols(kc=1, sl=1408)

BM = 2048
GRID = pl.cdiv(N, BM)


def _prep_body(degp_ref, x_ref, dinv_ref, xs_ref):
    deg = degp_ref[0] + degp_ref[1] + 1.0
    dinv = lax.rsqrt(deg)
    dinv_ref[...] = dinv
    xsc = x_ref[...] * dinv
    xs_ref[...] = jnp.concatenate([xsc, jnp.zeros_like(xsc)], axis=1)


def _mid_body(zp_ref, xs_ref, dinv_ref, w1_ref, b1_ref, w2_ref, gs_ref):
    dinv = dinv_ref[...]
    z = (zp_ref[0] + zp_ref[1] + xs_ref[...])[:, 0:4] * dinv
    h = jnp.dot(z, w1_ref[...], preferred_element_type=jnp.float32,
                precision=lax.Precision.HIGHEST)
    h = jnp.maximum(h + b1_ref[...], 0.0)
    g = jnp.dot(h, w2_ref[...], preferred_element_type=jnp.float32,
                precision=lax.Precision.HIGHEST)
    gs = g * dinv
    gs_ref[0] = gs[:, 0:8]
    gs_ref[1] = gs[:, 8:16]


def _post_body(qp_ref, gs_ref, dinv_ref, b2_ref, out_ref):
    q = jnp.concatenate([qp_ref[0] + gs_ref[0], qp_ref[1] + gs_ref[1]], axis=1)
    out_ref[...] = q * dinv_ref[...] + b2_ref[...]


def _prep_call(degp3, x):
    return pl.pallas_call(
        _prep_body,
        grid=(GRID,),
        in_specs=[pl.BlockSpec((NC, BM, 1), lambda i: (0, i, 0)),
                  pl.BlockSpec((BM, 4), lambda i: (i, 0))],
        out_specs=[pl.BlockSpec((BM, 1), lambda i: (i, 0)),
                   pl.BlockSpec((BM, 8), lambda i: (i, 0))],
        out_shape=[jax.ShapeDtypeStruct((N, 1), jnp.float32),
                   jax.ShapeDtypeStruct((NPAD, 8), jnp.float32)],
    )(degp3, x)


def _mid_call(zp, xs, dinv, w1, b1, w2):
    return pl.pallas_call(
        _mid_body,
        grid=(GRID,),
        in_specs=[pl.BlockSpec((NC, BM, 8), lambda i: (0, i, 0)),
                  pl.BlockSpec((BM, 8), lambda i: (i, 0)),
                  pl.BlockSpec((BM, 1), lambda i: (i, 0)),
                  pl.BlockSpec((4, 32), lambda i: (0, 0)),
                  pl.BlockSpec((1, 32), lambda i: (0, 0)),
                  pl.BlockSpec((32, 16), lambda i: (0, 0))],
        out_specs=pl.BlockSpec((NC, BM, 8), lambda i: (0, i, 0)),
        out_shape=jax.ShapeDtypeStruct((NC, NPAD, 8), jnp.float32),
    )(zp, xs, dinv, w1, b1, w2)


def _post_call(qp, gs, dinv, b2):
    return pl.pallas_call(
        _post_body,
        grid=(GRID,),
        in_specs=[pl.BlockSpec((NC, BM, 8), lambda i: (0, i, 0)),
                  pl.BlockSpec((NC, BM, 8), lambda i: (0, i, 0)),
                  pl.BlockSpec((BM, 1), lambda i: (i, 0)),
                  pl.BlockSpec((1, 16), lambda i: (0, 0))],
        out_specs=pl.BlockSpec((BM, 16), lambda i: (i, 0)),
        out_shape=jax.ShapeDtypeStruct((N, 16), jnp.float32),
    )(qp, gs, dinv, b2)


def kernel(x, edge_index, W1, b1, W2, b2):
    src = edge_index[0].astype(jnp.int32)
    dst = edge_index[1].astype(jnp.int32)
    pad = jnp.full((EP - E,), N, jnp.int32)
    src2 = jnp.concatenate([src, pad])
    dst2 = jnp.concatenate([dst, pad])
    ones = jnp.ones((1536,), jnp.float32)

    degp = _deg_call(dst2, ones, jnp.zeros((NB,), jnp.float32))
    dinv, xs = _prep_call(degp.reshape(NC, NPAD, 1), x)
    zp = _prop8_call(src2, dst2, xs, jnp.zeros((NB, 8), jnp.float32))
    gs2 = _mid_call(zp, xs, dinv, W1, b1.reshape(1, 32), W2)
    qp = _prop2_call(src2, dst2, gs2, jnp.zeros((NB, 8), jnp.float32))
    return _post_call(qp, gs2, dinv, b2.reshape(1, 16))


# speedup vs baseline: 98.4832x; 1.5716x over previous
import functools

import jax
import jax.numpy as jnp
from jax import lax
from jax.experimental import pallas as pl
from jax.experimental.pallas import tpu as pltpu
from jax.experimental.pallas import tpu_sc as plsc

N = 100000
E = 3200000
NC, NS = 2, 16
NW = NC * NS
LANE = 128
KC = 24
CHUNK = KC * LANE
EPW = ((E + NW * CHUNK - 1) // (NW * CHUNK)) * CHUNK
EP = EPW * NW
NCHUNKS = EPW // CHUNK
NB = 6272
NPAD = NB * NS

_MESH = plsc.VectorSubcoreMesh(core_axis_name="c", subcore_axis_name="s")


def _make_deg(kc, sl=LANE):
    chunk = kc * sl
    nchunks = EPW // chunk
    assert nchunks * chunk == EPW and nchunks % 2 == 0

    @functools.partial(
        pl.kernel,
        out_type=jax.ShapeDtypeStruct((NC * NPAD,), jnp.float32),
        mesh=_MESH,
        scratch_types=[
            pltpu.VMEM((chunk,), jnp.int32),
            pltpu.VMEM((chunk,), jnp.int32),
            pltpu.VMEM((sl,), jnp.float32),
            pltpu.VMEM_SHARED((NPAD,), jnp.float32),
            pltpu.SemaphoreType.DMA,
            pltpu.SemaphoreType.DMA,
        ],
        compiler_params=pltpu.CompilerParams(use_tc_tiling_on_sc=False),
    )
    def deg_kernel(dst_hbm, ones_hbm, zeros_hbm, out_hbm,
                   didx0, didx1, ones_v, acc, ss0, ss1):
        c = lax.axis_index("c")
        s = lax.axis_index("s")
        wid = c * NS + s
        didxs = (didx0, didx1)
        sss = (ss0, ss1)
        pltpu.sync_copy(ones_hbm, ones_v)
        pltpu.sync_copy(zeros_hbm, acc.at[pl.ds(s * NB, NB)])
        plsc.subcore_barrier()
        ebase0 = wid * EPW

        def scatter_descs(b):
            return [pltpu.make_async_copy(
                ones_v, acc.at[didxs[b].at[pl.ds(j * sl, sl)]], sss[b])
                for j in range(kc)]

        def body(i, carry):
            for b in range(2):
                @pl.when(i > 0)
                def _():
                    for d in scatter_descs(b):
                        d.wait()

                ebase = ebase0 + (2 * i + b) * chunk
                pltpu.sync_copy(dst_hbm.at[pl.ds(ebase, chunk)], didxs[b])
                for j in range(kc):
                    pltpu.async_copy(
                        ones_v, acc.at[didxs[b].at[pl.ds(j * sl, sl)]],
                        sss[b], add=True)
            return carry

        lax.fori_loop(0, nchunks // 2, body, 0)
        for b in range(2):
            for d in scatter_descs(b):
                d.wait()
        plsc.subcore_barrier()
        pltpu.sync_copy(acc.at[pl.ds(s * NB, NB)],
                        out_hbm.at[pl.ds(c * NPAD + s * NB, NB)])

    return deg_kernel


def _make_prop(C, stage_table, kc, sl=LANE):
    chunk = kc * sl
    nchunks = EPW // chunk
    assert nchunks * chunk == EPW and nchunks % 2 == 0
    scratch = [
        pltpu.VMEM((chunk,), jnp.int32),
        pltpu.VMEM((chunk,), jnp.int32),
        pltpu.VMEM((chunk,), jnp.int32),
        pltpu.VMEM((chunk,), jnp.int32),
        pltpu.VMEM((chunk, C), jnp.float32),
        pltpu.VMEM((chunk, C), jnp.float32),
        pltpu.VMEM_SHARED((NPAD, C), jnp.float32),
        pltpu.SemaphoreType.DMA,
        pltpu.SemaphoreType.DMA,
        pltpu.SemaphoreType.DMA,
        pltpu.SemaphoreType.DMA,
    ]
    if stage_table:
        scratch.append(pltpu.VMEM_SHARED((NPAD, C), jnp.float32))

    @functools.partial(
        pl.kernel,
        out_type=jax.ShapeDtypeStruct((NC * NPAD, C), jnp.float32),
        mesh=_MESH,
        scratch_types=scratch,
        compiler_params=pltpu.CompilerParams(use_tc_tiling_on_sc=False),
    )
    def prop_kernel(src_hbm, dst_hbm, table_hbm, zeros_hbm, out_hbm,
                    sidx0, sidx1, didx0, didx1, rows0, rows1, acc,
                    sg0, sg1, ss0, ss1, *maybe_tab):
        c = lax.axis_index("c")
        s = lax.axis_index("s")
        wid = c * NS + s
        sidxs, didxs = (sidx0, sidx1), (didx0, didx1)
        rowss, sgs, sss = (rows0, rows1), (sg0, sg1), (ss0, ss1)
        pltpu.sync_copy(zeros_hbm, acc.at[pl.ds(s * NB, NB)])
        if stage_table:
            table = maybe_tab[0]
            pltpu.sync_copy(table_hbm.at[pl.ds(s * NB, NB)],
                            table.at[pl.ds(s * NB, NB)])
        else:
            table = table_hbm
        plsc.subcore_barrier()
        ebase0 = wid * EPW

        def scatter_descs(b):
            return [pltpu.make_async_copy(
                rowss[b].at[pl.ds(j * sl, sl)],
                acc.at[didxs[b].at[pl.ds(j * sl, sl)]], sss[b])
                for j in range(kc)]

        def body(i, carry):
            for b in range(2):
                @pl.when(i > 0)
                def _():
                    for d in scatter_descs(b):
                        d.wait()

                ebase = ebase0 + (2 * i + b) * chunk
                pltpu.sync_copy(src_hbm.at[pl.ds(ebase, chunk)], sidxs[b])
                pltpu.sync_copy(dst_hbm.at[pl.ds(ebase, chunk)], didxs[b])
                for j in range(kc):
                    pltpu.async_copy(
                        table.at[sidxs[b].at[pl.ds(j * sl, sl)]],
                        rowss[b].at[pl.ds(j * sl, sl)], sgs[b])
                for j in range(kc):
                    pltpu.make_async_copy(
                        table.at[sidxs[b].at[pl.ds(j * sl, sl)]],
                        rowss[b].at[pl.ds(j * sl, sl)], sgs[b]).wait()
                for j in range(kc):
                    pltpu.async_copy(
                        rowss[b].at[pl.ds(j * sl, sl)],
                        acc.at[didxs[b].at[pl.ds(j * sl, sl)]],
                        sss[b], add=True)
            return carry

        lax.fori_loop(0, nchunks // 2, body, 0)
        for b in range(2):
            for d in scatter_descs(b):
                d.wait()
        plsc.subcore_barrier()
        pltpu.sync_copy(acc.at[pl.ds(s * NB, NB)],
                        out_hbm.at[pl.ds(c * NPAD + s * NB, NB)])

    return prop_kernel


def _make_prop_cols(kc, sl):
    chunk = kc * sl
    epc = EP // NS
    nchunks = epc // chunk
    assert nchunks * chunk == epc and nchunks % 2 == 0

    @functools.partial(
        pl.kernel,
        out_type=jax.ShapeDtypeStruct((NC * NPAD, 8), jnp.float32),
        mesh=_MESH,
        scratch_types=[
            pltpu.VMEM((chunk,), jnp.int32),
            pltpu.VMEM((chunk,), jnp.int32),
            pltpu.VMEM((chunk,), jnp.int32),
            pltpu.VMEM((chunk,), jnp.int32),
            pltpu.VMEM((chunk, 8), jnp.float32),
            pltpu.VMEM((chunk, 8), jnp.float32),
            pltpu.VMEM_SHARED((NPAD, 8), jnp.float32),
            pltpu.VMEM_SHARED((NPAD, 8), jnp.float32),
            pltpu.SemaphoreType.DMA,
            pltpu.SemaphoreType.DMA,
            pltpu.SemaphoreType.DMA,
            pltpu.SemaphoreType.DMA,
        ],
        compiler_params=pltpu.CompilerParams(use_tc_tiling_on_sc=False),
    )
    def prop_kernel(src_hbm, dst_hbm, table_hbm, zeros_hbm, out_hbm,
                    sidx0, sidx1, didx0, didx1, rows0, rows1, acc, table,
                    sg0, sg1, ss0, ss1):
        c = lax.axis_index("c")
        s = lax.axis_index("s")
        sidxs, didxs = (sidx0, sidx1), (didx0, didx1)
        rowss, sgs, sss = (rows0, rows1), (sg0, sg1), (ss0, ss1)
        pltpu.sync_copy(zeros_hbm, acc.at[pl.ds(s * NB, NB)])
        pltpu.sync_copy(table_hbm.at[c].at[pl.ds(s * NB, NB)],
                        table.at[pl.ds(s * NB, NB)])
        plsc.subcore_barrier()
        ebase0 = s * epc

        def scatter_descs(b):
            return [pltpu.make_async_copy(
                rowss[b].at[pl.ds(j * sl, sl)],
                acc.at[didxs[b].at[pl.ds(j * sl, sl)]], sss[b])
                for j in range(kc)]

        def body(i, carry):
            for b in range(2):
                @pl.when(i > 0)
                def _():
                    for d in scatter_descs(b):
                        d.wait()

                ebase = ebase0 + (2 * i + b) * chunk
                pltpu.sync_copy(src_hbm.at[pl.ds(ebase, chunk)], sidxs[b])
                pltpu.sync_copy(dst_hbm.at[pl.ds(ebase, chunk)], didxs[b])
                for j in range(kc):
                    pltpu.async_copy(
                        table.at[sidxs[b].at[pl.ds(j * sl, sl)]],
                        rowss[b].at[pl.ds(j * sl, sl)], sgs[b])
                for j in range(kc):
                    pltpu.make_async_copy(
                        table.at[sidxs[b].at[pl.ds(j * sl, sl)]],
                        rowss[b].at[pl.ds(j * sl, sl)], sgs[b]).wait()
                for j in range(kc):
                    pltpu.async_copy(
                        rowss[b].at[pl.ds(j * sl, sl)],
                        acc.at[didxs[b].at[pl.ds(j * sl, sl)]],
                        sss[b], add=True)
            return carry

        lax.fori_loop(0, nchunks // 2, body, 0)
        for b in range(2):
            for d in scatter_descs(b):
                d.wait()
        plsc.subcore_barrier()
        pltpu.sync_copy(acc.at[pl.ds(s * NB, NB)],
                        out_hbm.at[pl.ds(c * NPAD + s * NB, NB)])

    return prop_kernel


_deg_call = _make_deg(kc=1, sl=1536)
_prop8_call = _make_prop(8, stage_table=True, kc=1, sl=1408)
_prop2_call = _make_prop_cols(kc=1, sl=1408)

RP = NPAD // LANE
RX = NPAD * 8 // LANE
BMP = 112
BMR = 392


def _prep_body(p0_ref, p1_ref, dinv_ref):
    dinv_ref[...] = lax.rsqrt(p0_ref[...] + p1_ref[...] + 1.0)


def _xs_body(xp_ref, dinv8_ref, xs_ref):
    xs_ref[...] = xp_ref[...] * dinv8_ref[...]


def _mid_body(z0_ref, z1_ref, xs_ref, dinv8_ref, w1b_ref, b1il_ref,
              w2a_ref, w2b_ref, g2_ref):
    dinv8 = dinv8_ref[...]
    z = (z0_ref[...] + z1_ref[...] + xs_ref[...]) * dinv8
    h = jnp.dot(z, w1b_ref[...], preferred_element_type=jnp.float32,
                precision=lax.Precision.HIGHEST)
    h = jnp.maximum(h + b1il_ref[...], 0.0)
    g2_ref[0] = jnp.dot(h, w2a_ref[...], preferred_element_type=jnp.float32,
                        precision=lax.Precision.HIGHEST) * dinv8
    g2_ref[1] = jnp.dot(h, w2b_ref[...], preferred_element_type=jnp.float32,
                        precision=lax.Precision.HIGHEST) * dinv8


def _post_body(q0_ref, q1_ref, g0_ref, g1_ref, dinv8_ref, b2a_ref, b2b_ref,
               o2_ref):
    dinv8 = dinv8_ref[...]
    o2_ref[0] = (q0_ref[...] + g0_ref[...]) * dinv8 + b2a_ref[...]
    o2_ref[1] = (q1_ref[...] + g1_ref[...]) * dinv8 + b2b_ref[...]


def _prep_call(degf2):
    return pl.pallas_call(
        _prep_body,
        grid=(RP // BMP,),
        in_specs=[pl.BlockSpec((BMP, LANE), lambda i: (i, 0)),
                  pl.BlockSpec((BMP, LANE), lambda i: (i + RP // BMP, 0))],
        out_specs=pl.BlockSpec((BMP, LANE), lambda i: (i, 0)),
        out_shape=jax.ShapeDtypeStruct((RP, LANE), jnp.float32),
    )(degf2, degf2)


def _xs_call(xp_il, dinv8):
    return pl.pallas_call(
        _xs_body,
        grid=(RX // BMR,),
        in_specs=[pl.BlockSpec((BMR, LANE), lambda i: (i, 0)),
                  pl.BlockSpec((BMR, LANE), lambda i: (i, 0))],
        out_specs=pl.BlockSpec((BMR, LANE), lambda i: (i, 0)),
        out_shape=jax.ShapeDtypeStruct((RX, LANE), jnp.float32),
    )(xp_il, dinv8)


def _mid_call(zpf2, xs_il, dinv8, w1b, b1il, w2a, w2b):
    nblk = RX // BMR
    return pl.pallas_call(
        _mid_body,
        grid=(nblk,),
        in_specs=[pl.BlockSpec((BMR, LANE), lambda i: (i, 0)),
                  pl.BlockSpec((BMR, LANE), lambda i, n=nblk: (i + n, 0)),
                  pl.BlockSpec((BMR, LANE), lambda i: (i, 0)),
                  pl.BlockSpec((BMR, LANE), lambda i: (i, 0)),
                  pl.BlockSpec((LANE, 512), lambda i: (0, 0)),
                  pl.BlockSpec((1, 512), lambda i: (0, 0)),
                  pl.BlockSpec((512, LANE), lambda i: (0, 0)),
                  pl.BlockSpec((512, LANE), lambda i: (0, 0))],
        out_specs=pl.BlockSpec((NC, BMR, LANE), lambda i: (0, i, 0)),
        out_shape=jax.ShapeDtypeStruct((NC, RX, LANE), jnp.float32),
    )(zpf2, zpf2, xs_il, dinv8, w1b, b1il, w2a, w2b)


def _post_call(qpf2, g2f, dinv8, b2a, b2b):
    nblk = RX // BMR
    return pl.pallas_call(
        _post_body,
        grid=(nblk,),
        in_specs=[pl.BlockSpec((BMR, LANE), lambda i: (i, 0)),
                  pl.BlockSpec((BMR, LANE), lambda i, n=nblk: (i + n, 0)),
                  pl.BlockSpec((BMR, LANE), lambda i: (i, 0)),
                  pl.BlockSpec((BMR, LANE), lambda i, n=nblk: (i + n, 0)),
                  pl.BlockSpec((BMR, LANE), lambda i: (i, 0)),
                  pl.BlockSpec((1, LANE), lambda i: (0, 0)),
                  pl.BlockSpec((1, LANE), lambda i: (0, 0))],
        out_specs=pl.BlockSpec((NC, BMR, LANE), lambda i: (0, i, 0)),
        out_shape=jax.ShapeDtypeStruct((NC, RX, LANE), jnp.float32),
    )(qpf2, qpf2, g2f, g2f, dinv8, b2a, b2b)


def kernel(x, edge_index, W1, b1, W2, b2):
    src = edge_index[0].astype(jnp.int32)
    dst = edge_index[1].astype(jnp.int32)
    pad = jnp.full((EP - E,), N, jnp.int32)
    src2 = jnp.concatenate([src, pad])
    dst2 = jnp.concatenate([dst, pad])
    ones = jnp.ones((1536,), jnp.float32)
    eye16 = jnp.eye(16, dtype=jnp.float32)
    w1b = jnp.kron(eye16, jnp.pad(W1, ((0, 4), (0, 0))))
    b1il = jnp.tile(b1, 16).reshape(1, 512)
    w2a = jnp.kron(eye16, W2[:, 0:8])
    w2b = jnp.kron(eye16, W2[:, 8:16])
    b2a = jnp.tile(b2[0:8], 16).reshape(1, LANE)
    b2b = jnp.tile(b2[8:16], 16).reshape(1, LANE)

    degf = _deg_call(dst2, ones, jnp.zeros((NB,), jnp.float32))
    dinv = _prep_call(degf.reshape(NC * RP, LANE))
    dinv8 = jnp.broadcast_to(dinv.reshape(NPAD, 1),
                             (NPAD, 8)).reshape(RX, LANE)
    xp_il = jnp.pad(x, ((0, NPAD - N), (0, 4))).reshape(RX, LANE)
    xs_il = _xs_call(xp_il, dinv8)
    zpf = _prop8_call(src2, dst2, xs_il.reshape(NPAD, 8),
                      jnp.zeros((NB, 8), jnp.float32))
    g2 = _mid_call(zpf.reshape(NC * RX, LANE), xs_il, dinv8,
                   w1b, b1il, w2a, w2b)
    qpf = _prop2_call(src2, dst2, g2.reshape(NC, NPAD, 8),
                      jnp.zeros((NB, 8), jnp.float32))
    o2 = _post_call(qpf.reshape(NC * RX, LANE), g2.reshape(NC * RX, LANE),
                    dinv8, b2a, b2b)
    oa = o2[0].reshape(NPAD, 8)[:N]
    ob = o2[1].reshape(NPAD, 8)[:N]
    return jnp.concatenate([oa, ob], axis=1)


# speedup vs baseline: 100.2689x; 1.0181x over previous
import functools

import jax
import jax.numpy as jnp
from jax import lax
from jax.experimental import pallas as pl
from jax.experimental.pallas import tpu as pltpu
from jax.experimental.pallas import tpu_sc as plsc

N = 100000
E = 3200000
NC, NS = 2, 16
NW = NC * NS
LANE = 128
KC = 24
CHUNK = KC * LANE
EPW = ((E + NW * CHUNK - 1) // (NW * CHUNK)) * CHUNK
EP = EPW * NW
NCHUNKS = EPW // CHUNK
NB = 6272
NPAD = NB * NS

_MESH = plsc.VectorSubcoreMesh(core_axis_name="c", subcore_axis_name="s")


def _make_deg(kc, sl=LANE):
    chunk = kc * sl
    nchunks = EPW // chunk
    assert nchunks * chunk == EPW and nchunks % 2 == 0

    @functools.partial(
        pl.kernel,
        out_type=jax.ShapeDtypeStruct((NC * NPAD,), jnp.float32),
        mesh=_MESH,
        scratch_types=[
            pltpu.VMEM((chunk,), jnp.int32),
            pltpu.VMEM((chunk,), jnp.int32),
            pltpu.VMEM((sl,), jnp.float32),
            pltpu.VMEM_SHARED((NPAD,), jnp.float32),
            pltpu.SemaphoreType.DMA,
            pltpu.SemaphoreType.DMA,
        ],
        compiler_params=pltpu.CompilerParams(use_tc_tiling_on_sc=False),
    )
    def deg_kernel(dst_hbm, ones_hbm, zeros_hbm, out_hbm,
                   didx0, didx1, ones_v, acc, ss0, ss1):
        c = lax.axis_index("c")
        s = lax.axis_index("s")
        wid = c * NS + s
        didxs = (didx0, didx1)
        sss = (ss0, ss1)
        pltpu.sync_copy(ones_hbm, ones_v)
        pltpu.sync_copy(zeros_hbm, acc.at[pl.ds(s * NB, NB)])
        plsc.subcore_barrier()
        ebase0 = wid * EPW

        def scatter_descs(b):
            return [pltpu.make_async_copy(
                ones_v, acc.at[didxs[b].at[pl.ds(j * sl, sl)]], sss[b])
                for j in range(kc)]

        def body(i, carry):
            for b in range(2):
                @pl.when(i > 0)
                def _():
                    for d in scatter_descs(b):
                        d.wait()

                ebase = ebase0 + (2 * i + b) * chunk
                pltpu.sync_copy(dst_hbm.at[pl.ds(ebase, chunk)], didxs[b])
                for j in range(kc):
                    pltpu.async_copy(
                        ones_v, acc.at[didxs[b].at[pl.ds(j * sl, sl)]],
                        sss[b], add=True)
            return carry

        lax.fori_loop(0, nchunks // 2, body, 0)
        for b in range(2):
            for d in scatter_descs(b):
                d.wait()
        plsc.subcore_barrier()
        pltpu.sync_copy(acc.at[pl.ds(s * NB, NB)],
                        out_hbm.at[pl.ds(c * NPAD + s * NB, NB)])

    return deg_kernel


def _make_prop(C, stage_table, kc, sl=LANE):
    chunk = kc * sl
    nchunks = EPW // chunk
    assert nchunks * chunk == EPW and nchunks % 2 == 0
    scratch = [
        pltpu.VMEM((chunk,), jnp.int32),
        pltpu.VMEM((chunk,), jnp.int32),
        pltpu.VMEM((chunk,), jnp.int32),
        pltpu.VMEM((chunk,), jnp.int32),
        pltpu.VMEM((chunk, C), jnp.float32),
        pltpu.VMEM((chunk, C), jnp.float32),
        pltpu.VMEM_SHARED((NPAD, C), jnp.float32),
        pltpu.SemaphoreType.DMA,
        pltpu.SemaphoreType.DMA,
        pltpu.SemaphoreType.DMA,
        pltpu.SemaphoreType.DMA,
    ]
    if stage_table:
        scratch.append(pltpu.VMEM_SHARED((NPAD, C), jnp.float32))

    @functools.partial(
        pl.kernel,
        out_type=jax.ShapeDtypeStruct((NC * NPAD, C), jnp.float32),
        mesh=_MESH,
        scratch_types=scratch,
        compiler_params=pltpu.CompilerParams(use_tc_tiling_on_sc=False),
    )
    def prop_kernel(src_hbm, dst_hbm, table_hbm, zeros_hbm, out_hbm,
                    sidx0, sidx1, didx0, didx1, rows0, rows1, acc,
                    sg0, sg1, ss0, ss1, *maybe_tab):
        c = lax.axis_index("c")
        s = lax.axis_index("s")
        wid = c * NS + s
        sidxs, didxs = (sidx0, sidx1), (didx0, didx1)
        rowss, sgs, sss = (rows0, rows1), (sg0, sg1), (ss0, ss1)
        pltpu.sync_copy(zeros_hbm, acc.at[pl.ds(s * NB, NB)])
        if stage_table:
            table = maybe_tab[0]
            pltpu.sync_copy(table_hbm.at[pl.ds(s * NB, NB)],
                            table.at[pl.ds(s * NB, NB)])
        else:
            table = table_hbm
        plsc.subcore_barrier()
        ebase0 = wid * EPW

        def scatter_descs(b):
            return [pltpu.make_async_copy(
                rowss[b].at[pl.ds(j * sl, sl)],
                acc.at[didxs[b].at[pl.ds(j * sl, sl)]], sss[b])
                for j in range(kc)]

        def body(i, carry):
            for b in range(2):
                @pl.when(i > 0)
                def _():
                    for d in scatter_descs(b):
                        d.wait()

                ebase = ebase0 + (2 * i + b) * chunk
                pltpu.sync_copy(src_hbm.at[pl.ds(ebase, chunk)], sidxs[b])
                pltpu.sync_copy(dst_hbm.at[pl.ds(ebase, chunk)], didxs[b])
                for j in range(kc):
                    pltpu.async_copy(
                        table.at[sidxs[b].at[pl.ds(j * sl, sl)]],
                        rowss[b].at[pl.ds(j * sl, sl)], sgs[b])
                for j in range(kc):
                    pltpu.make_async_copy(
                        table.at[sidxs[b].at[pl.ds(j * sl, sl)]],
                        rowss[b].at[pl.ds(j * sl, sl)], sgs[b]).wait()
                for j in range(kc):
                    pltpu.async_copy(
                        rowss[b].at[pl.ds(j * sl, sl)],
                        acc.at[didxs[b].at[pl.ds(j * sl, sl)]],
                        sss[b], add=True)
            return carry

        lax.fori_loop(0, nchunks // 2, body, 0)
        for b in range(2):
            for d in scatter_descs(b):
                d.wait()
        plsc.subcore_barrier()
        pltpu.sync_copy(acc.at[pl.ds(s * NB, NB)],
                        out_hbm.at[pl.ds(c * NPAD + s * NB, NB)])

    return prop_kernel


def _make_prop_cols(kc, sl):
    chunk = kc * sl
    epc = EP // NS
    nchunks = epc // chunk
    assert nchunks * chunk == epc and nchunks % 2 == 0

    @functools.partial(
        pl.kernel,
        out_type=jax.ShapeDtypeStruct((NC * NPAD, 8), jnp.float32),
        mesh=_MESH,
        scratch_types=[
            pltpu.VMEM((chunk,), jnp.int32),
            pltpu.VMEM((chunk,), jnp.int32),
            pltpu.VMEM((chunk,), jnp.int32),
            pltpu.VMEM((chunk,), jnp.int32),
            pltpu.VMEM((chunk, 8), jnp.float32),
            pltpu.VMEM((chunk, 8), jnp.float32),
            pltpu.VMEM_SHARED((NPAD, 8), jnp.float32),
            pltpu.VMEM_SHARED((NPAD, 8), jnp.float32),
            pltpu.SemaphoreType.DMA,
            pltpu.SemaphoreType.DMA,
            pltpu.SemaphoreType.DMA,
            pltpu.SemaphoreType.DMA,
        ],
        compiler_params=pltpu.CompilerParams(use_tc_tiling_on_sc=False),
    )
    def prop_kernel(src_hbm, dst_hbm, table_hbm, zeros_hbm, out_hbm,
                    sidx0, sidx1, didx0, didx1, rows0, rows1, acc, table,
                    sg0, sg1, ss0, ss1):
        c = lax.axis_index("c")
        s = lax.axis_index("s")
        sidxs, didxs = (sidx0, sidx1), (didx0, didx1)
        rowss, sgs, sss = (rows0, rows1), (sg0, sg1), (ss0, ss1)
        pltpu.sync_copy(zeros_hbm, acc.at[pl.ds(s * NB, NB)])
        pltpu.sync_copy(table_hbm.at[c].at[pl.ds(s * NB, NB)],
                        table.at[pl.ds(s * NB, NB)])
        plsc.subcore_barrier()
        ebase0 = s * epc

        def scatter_descs(b):
            return [pltpu.make_async_copy(
                rowss[b].at[pl.ds(j * sl, sl)],
                acc.at[didxs[b].at[pl.ds(j * sl, sl)]], sss[b])
                for j in range(kc)]

        def body(i, carry):
            for b in range(2):
                @pl.when(i > 0)
                def _():
                    for d in scatter_descs(b):
                        d.wait()

                ebase = ebase0 + (2 * i + b) * chunk
                pltpu.sync_copy(src_hbm.at[pl.ds(ebase, chunk)], sidxs[b])
                pltpu.sync_copy(dst_hbm.at[pl.ds(ebase, chunk)], didxs[b])
                for j in range(kc):
                    pltpu.async_copy(
                        table.at[sidxs[b].at[pl.ds(j * sl, sl)]],
                        rowss[b].at[pl.ds(j * sl, sl)], sgs[b])
                for j in range(kc):
                    pltpu.make_async_copy(
                        table.at[sidxs[b].at[pl.ds(j * sl, sl)]],
                        rowss[b].at[pl.ds(j * sl, sl)], sgs[b]).wait()
                for j in range(kc):
                    pltpu.async_copy(
                        rowss[b].at[pl.ds(j * sl, sl)],
                        acc.at[didxs[b].at[pl.ds(j * sl, sl)]],
                        sss[b], add=True)
            return carry

        lax.fori_loop(0, nchunks // 2, body, 0)
        for b in range(2):
            for d in scatter_descs(b):
                d.wait()
        plsc.subcore_barrier()
        pltpu.sync_copy(acc.at[pl.ds(s * NB, NB)],
                        out_hbm.at[pl.ds(c * NPAD + s * NB, NB)])

    return prop_kernel


_deg_call = _make_deg(kc=1, sl=1536)
_prop8_call = _make_prop(8, stage_table=True, kc=1, sl=1408)
_prop2_call = _make_prop_cols(kc=1, sl=1408)

RP = NPAD // LANE
RX = NPAD * 8 // LANE
BMP = 112
BMR = 392


def _prep_body(p0_ref, p1_ref, dinv_ref):
    dinv_ref[...] = lax.rsqrt(p0_ref[...] + p1_ref[...] + 1.0)


def _xs_body(xp_ref, dinv8_ref, xs_ref):
    xs_ref[...] = xp_ref[...] * dinv8_ref[...]


def _mid_body(z0_ref, z1_ref, xs_ref, dinv8_ref, w1b_ref, b1il_ref,
              w2a_ref, w2b_ref, g2_ref):
    dinv8 = dinv8_ref[...]
    z = (z0_ref[...] + z1_ref[...] + xs_ref[...]) * dinv8
    h = jnp.dot(z, w1b_ref[...], preferred_element_type=jnp.float32,
                precision=lax.Precision.HIGHEST)
    h = jnp.maximum(h + b1il_ref[...], 0.0)
    g2_ref[0] = jnp.dot(h, w2a_ref[...], preferred_element_type=jnp.float32,
                        precision=lax.Precision.HIGHEST) * dinv8
    g2_ref[1] = jnp.dot(h, w2b_ref[...], preferred_element_type=jnp.float32,
                        precision=lax.Precision.HIGHEST) * dinv8


def _post_body(q0_ref, q1_ref, g0_ref, g1_ref, dinv8_ref, b2a_ref, b2b_ref,
               o2_ref):
    dinv8 = dinv8_ref[...]
    o2_ref[0] = (q0_ref[...] + g0_ref[...]) * dinv8 + b2a_ref[...]
    o2_ref[1] = (q1_ref[...] + g1_ref[...]) * dinv8 + b2b_ref[...]


def _prep_call(degf2):
    return pl.pallas_call(
        _prep_body,
        grid=(RP // BMP,),
        in_specs=[pl.BlockSpec((BMP, LANE), lambda i: (i, 0)),
                  pl.BlockSpec((BMP, LANE), lambda i: (i + RP // BMP, 0))],
        out_specs=pl.BlockSpec((BMP, LANE), lambda i: (i, 0)),
        out_shape=jax.ShapeDtypeStruct((RP, LANE), jnp.float32),
    )(degf2, degf2)


def _xs_call(xp_il, dinv8):
    return pl.pallas_call(
        _xs_body,
        grid=(RX // BMR,),
        in_specs=[pl.BlockSpec((BMR, LANE), lambda i: (i, 0)),
                  pl.BlockSpec((BMR, LANE), lambda i: (i, 0))],
        out_specs=pl.BlockSpec((BMR, LANE), lambda i: (i, 0)),
        out_shape=jax.ShapeDtypeStruct((RX, LANE), jnp.float32),
    )(xp_il, dinv8)


def _mid_call(zpf2, xs_il, dinv8, w1b, b1il, w2a, w2b):
    nblk = RX // BMR
    return pl.pallas_call(
        _mid_body,
        grid=(nblk,),
        in_specs=[pl.BlockSpec((BMR, LANE), lambda i: (i, 0)),
                  pl.BlockSpec((BMR, LANE), lambda i, n=nblk: (i + n, 0)),
                  pl.BlockSpec((BMR, LANE), lambda i: (i, 0)),
                  pl.BlockSpec((BMR, LANE), lambda i: (i, 0)),
                  pl.BlockSpec((LANE, 512), lambda i: (0, 0)),
                  pl.BlockSpec((1, 512), lambda i: (0, 0)),
                  pl.BlockSpec((512, LANE), lambda i: (0, 0)),
                  pl.BlockSpec((512, LANE), lambda i: (0, 0))],
        out_specs=pl.BlockSpec((NC, BMR, LANE), lambda i: (0, i, 0)),
        out_shape=jax.ShapeDtypeStruct((NC, RX, LANE), jnp.float32),
    )(zpf2, zpf2, xs_il, dinv8, w1b, b1il, w2a, w2b)


def _post_call(qpf2, g2f, dinv8, b2a, b2b):
    nblk = RX // BMR
    return pl.pallas_call(
        _post_body,
        grid=(nblk,),
        in_specs=[pl.BlockSpec((BMR, LANE), lambda i: (i, 0)),
                  pl.BlockSpec((BMR, LANE), lambda i, n=nblk: (i + n, 0)),
                  pl.BlockSpec((BMR, LANE), lambda i: (i, 0)),
                  pl.BlockSpec((BMR, LANE), lambda i, n=nblk: (i + n, 0)),
                  pl.BlockSpec((BMR, LANE), lambda i: (i, 0)),
                  pl.BlockSpec((1, LANE), lambda i: (0, 0)),
                  pl.BlockSpec((1, LANE), lambda i: (0, 0))],
        out_specs=pl.BlockSpec((NC, BMR, LANE), lambda i: (0, i, 0)),
        out_shape=jax.ShapeDtypeStruct((NC, RX, LANE), jnp.float32),
    )(qpf2, qpf2, g2f, g2f, dinv8, b2a, b2b)


def kernel(x, edge_index, W1, b1, W2, b2):
    src = edge_index[0].astype(jnp.int32)
    dst = edge_index[1].astype(jnp.int32)
    pad = jnp.full((EP - E,), N, jnp.int32)
    src2 = jnp.concatenate([src, pad])
    dst2 = jnp.concatenate([dst, pad])
    ones = jnp.ones((1536,), jnp.float32)
    eye16 = jnp.eye(16, dtype=jnp.float32)
    w1b = jnp.kron(eye16, jnp.pad(W1, ((0, 4), (0, 0))))
    b1il = jnp.tile(b1, 16).reshape(1, 512)
    w2a = jnp.kron(eye16, W2[:, 0:8])
    w2b = jnp.kron(eye16, W2[:, 8:16])
    b2a = jnp.tile(b2[0:8], 16).reshape(1, LANE)
    b2b = jnp.tile(b2[8:16], 16).reshape(1, LANE)

    degf = _deg_call(dst2, ones, jnp.zeros((NB,), jnp.float32))
    dinv = _prep_call(degf.reshape(NC * RP, LANE))
    dinv8 = jnp.broadcast_to(dinv.reshape(NPAD, 1),
                             (NPAD, 8)).reshape(RX, LANE)
    xp_il = jnp.pad(x, ((0, NPAD - N), (0, 4))).reshape(RX, LANE)
    xs_il = _xs_call(xp_il, dinv8)
    zpf = _prop8_call(src2, dst2, xs_il.reshape(NPAD, 8),
                      jnp.zeros((NB, 8), jnp.float32))
    g2 = _mid_call(zpf.reshape(NC * RX, LANE), xs_il, dinv8,
                   w1b, b1il, w2a, w2b)
    qpf = _prop2_call(src2, dst2, g2.reshape(NC, NPAD, 8),
                      jnp.zeros((NB, 8), jnp.float32))
    o2 = _post_call(qpf.reshape(NC * RX, LANE), g2.reshape(NC * RX, LANE),
                    dinv8, b2a, b2b)
    full = jnp.concatenate([o2[0].reshape(NPAD, 8), o2[1].reshape(NPAD, 8)],
                           axis=1)
    return lax.slice(full, (0, 0), (N, 16))
